# baseline (speedup 1.0000x reference)
import jax
import jax.numpy as jnp
from jax import lax
from jax.experimental import pallas as pl
from jax.experimental.pallas import tpu as pltpu
from jax.experimental.pallas import tpu_sc as plsc

N = 10000
NP = 10240
E = 320000

NS = 16
NC = 2
EPT = E // NS
K = 40
NCHUNK = EPT // K
RPT = NP // NS

_mesh = plsc.VectorSubcoreMesh(core_axis_name="c", subcore_axis_name="s")


def _deg_body(edges, degs, isrc_v, idst_v, dsrc_v, ddst_v, pbuf_v, res_v, part_sh):
    cid = lax.axis_index("c")
    sid = lax.axis_index("s")
    zeros16 = jnp.zeros((16,), jnp.float32)
    ones16 = jnp.ones((16,), jnp.float32)

    def zero_body(i, _):
        dsrc_v[pl.ds(i * 16, 16)] = zeros16
        ddst_v[pl.ds(i * 16, 16)] = zeros16
        return 0

    lax.fori_loop(0, NP // 16, zero_body, 0)

    base = cid * (2 * E) + sid * EPT

    def chunk_body(j, _):
        off = base + j * K
        pltpu.sync_copy(edges.at[pl.ds(off, K)], isrc_v)
        pltpu.sync_copy(edges.at[pl.ds(off + E, K)], idst_v)
        for k in range(K // 16):
            i16 = isrc_v[pl.ds(k * 16, 16)]
            plsc.addupdate_scatter(dsrc_v, [i16], ones16)
            j16 = idst_v[pl.ds(k * 16, 16)]
            plsc.addupdate_scatter(ddst_v, [j16], ones16)
        return 0

    lax.fori_loop(0, NCHUNK, chunk_body, 0)

    pltpu.sync_copy(dsrc_v, part_sh.at[0, sid])
    pltpu.sync_copy(ddst_v, part_sh.at[1, sid])
    plsc.subcore_barrier()

    r0 = sid * RPT
    for t in range(2):
        for p in range(NS):
            pltpu.sync_copy(part_sh.at[t, p, pl.ds(r0, RPT)], pbuf_v.at[p])

        def red_loop(q, _):
            s = pbuf_v[0, pl.ds(q * 16, 16)]
            for p in range(1, NS):
                s = s + pbuf_v[p, pl.ds(q * 16, 16)]
            res_v[pl.ds(q * 16, 16)] = s
            return 0

        lax.fori_loop(0, RPT // 16, red_loop, 0)
        pltpu.sync_copy(res_v, degs.at[pl.ds((cid * 2 + t) * NP + r0, RPT)])


_NBUF = 5
_LA = 4


def _agg_body(hs, srcoff, dst, zrows, agg, isrc_v,
              i0, i1, i2, i3, i4, rows_v,
              g0, g1, g2, g3, g4, s0, s1, s2, s3, s4, agg_sh):
    cid = lax.axis_index("c")
    sid = lax.axis_index("s")

    r0 = sid * RPT
    pltpu.sync_copy(zrows.at[pl.ds(r0, RPT)], agg_sh.at[pl.ds(r0, RPT)])

    sbase = cid * E + sid * EPT
    pltpu.sync_copy(srcoff.at[pl.ds(sbase, EPT)], isrc_v)
    plsc.subcore_barrier()

    idsts = (i0, i1, i2, i3, i4)
    gsems = (g0, g1, g2, g3, g4)
    ssems = (s0, s1, s2, s3, s4)

    def stage_fire(j, b):
        pltpu.sync_copy(dst.at[pl.ds(sbase + j * K, K)], idsts[b])
        pltpu.async_copy(hs.at[isrc_v.at[pl.ds(j * K, K)]], rows_v.at[b],
                         gsems[b])

    for b in range(_LA):
        stage_fire(b, b)

    def outer(jj, _):
        for b in range(_NBUF):
            j = jj * _NBUF + b
            pltpu.make_async_copy(hs.at[isrc_v.at[pl.ds(j * K, K)]],
                                  rows_v.at[b], gsems[b]).wait()
            pltpu.async_copy(rows_v.at[b], agg_sh.at[idsts[b]], ssems[b],
                             add=True)
            b4 = (b + _LA) % _NBUF

            @pl.when(j + _LA < NCHUNK)
            def _():
                @pl.when(j >= 1)
                def _():
                    pltpu.make_async_copy(rows_v.at[b4],
                                          agg_sh.at[idsts[b4]],
                                          ssems[b4]).wait()
                stage_fire(j + _LA, b4)
        return 0

    lax.fori_loop(0, NCHUNK // _NBUF, outer, 0)
    for b in range(_NBUF):
        pltpu.make_async_copy(rows_v.at[b], agg_sh.at[idsts[b]],
                              ssems[b]).wait()
    plsc.subcore_barrier()
    pltpu.sync_copy(agg_sh.at[pl.ds(r0, RPT)], agg.at[pl.ds(cid * NP + r0, RPT)])


def _make_deg_kernel():
    return pl.kernel(
        _deg_body,
        out_type=jax.ShapeDtypeStruct((4 * NP,), jnp.float32),
        mesh=_mesh,
        compiler_params=pltpu.CompilerParams(needs_layout_passes=False),
        scratch_types=[
            pltpu.VMEM((K,), jnp.int32),
            pltpu.VMEM((K,), jnp.int32),
            pltpu.VMEM((NP,), jnp.float32),
            pltpu.VMEM((NP,), jnp.float32),
            pltpu.VMEM((NS, RPT), jnp.float32),
            pltpu.VMEM((RPT,), jnp.float32),
            pltpu.VMEM_SHARED((2, NS, NP), jnp.float32),
        ],
    )


def _make_agg_kernel(fh):
    return pl.kernel(
        _agg_body,
        out_type=jax.ShapeDtypeStruct((NC * NP, fh), jnp.float32),
        mesh=_mesh,
        compiler_params=pltpu.CompilerParams(needs_layout_passes=False),
        scratch_types=(
            [pltpu.VMEM((EPT,), jnp.int32)]
            + [pltpu.VMEM((K,), jnp.int32) for _ in range(_NBUF)]
            + [pltpu.VMEM((_NBUF, K, fh), jnp.float32)]
            + [pltpu.SemaphoreType.DMA for _ in range(2 * _NBUF)]
            + [pltpu.VMEM_SHARED((NP, fh), jnp.float32)]
        ),
    )


def _norms_kernel(deg_ref, out_ref):
    d = deg_ref[...]
    out_ref[...] = lax.rsqrt(jnp.maximum(d, 1.0))


def _dense0_kernel(x_ref, w_ref, ns_ref, out_ref):
    x = x_ref[...]
    w = w_ref[...]
    h = jnp.dot(x, w, preferred_element_type=jnp.float32)
    out_ref[...] = h * ns_ref[...]


def _dense_kernel(a0_ref, a1_ref, nd_ref, b_ref, w_ref, ns_ref, out_ref):
    x = jnp.concatenate([a0_ref[...], a1_ref[...]], axis=-1)
    x = jnp.maximum(x * nd_ref[...] + b_ref[...], 0.0)
    h = jnp.dot(x, w_ref[...], preferred_element_type=jnp.float32)
    out_ref[...] = h * ns_ref[...]


def _final_kernel(a1_ref, a2_ref, nd1_ref, nd2_ref, b_ref, m_ref, out_ref):
    o1 = a1_ref[...] * nd1_ref[...] + b_ref[...]
    o2 = a2_ref[...] * nd2_ref[...] + b_ref[...]
    out_ref[...] = jnp.where(m_ref[...] == 0, o1, o2)


_BLK = 640
_NB = NP // _BLK


def _tc_norms(degs):
    return pl.pallas_call(
        _norms_kernel,
        out_shape=jax.ShapeDtypeStruct((4, NP), jnp.float32),
    )(degs.reshape(4, NP))


def _tc_dense0(x, w, ns):
    fh = w.shape[1] // 2
    return pl.pallas_call(
        _dense0_kernel,
        grid=(_NB, 2),
        in_specs=[
            pl.BlockSpec((_BLK, 128), lambda i, c: (i, 0)),
            pl.BlockSpec((128, fh), lambda i, c: (0, c)),
            pl.BlockSpec((_BLK, 1), lambda i, c: (i, 0)),
        ],
        out_specs=pl.BlockSpec((_BLK, fh), lambda i, c: (c * _NB + i, 0)),
        out_shape=jax.ShapeDtypeStruct((2 * NP, fh), jnp.float32),
    )(x, w, ns)


def _tc_dense(agg, nd, b, w, ns):
    fh = w.shape[1] // 2
    return pl.pallas_call(
        _dense_kernel,
        grid=(_NB, 2),
        in_specs=[
            pl.BlockSpec((_BLK, 128), lambda i, c: (i, 0)),
            pl.BlockSpec((_BLK, 128), lambda i, c: (_NB + i, 0)),
            pl.BlockSpec((_BLK, 1), lambda i, c: (i, 0)),
            pl.BlockSpec((1, 256), lambda i, c: (0, 0)),
            pl.BlockSpec((256, fh), lambda i, c: (0, c)),
            pl.BlockSpec((_BLK, 1), lambda i, c: (i, 0)),
        ],
        out_specs=pl.BlockSpec((_BLK, fh), lambda i, c: (c * _NB + i, 0)),
        out_shape=jax.ShapeDtypeStruct((2 * NP, fh), jnp.float32),
    )(agg, agg, nd, b, w, ns)


def _tc_dense_last(agg, nd, b, w, ns):
    return pl.pallas_call(
        _dense_kernel,
        grid=(_NB,),
        in_specs=[
            pl.BlockSpec((_BLK, 128), lambda i: (i, 0)),
            pl.BlockSpec((_BLK, 128), lambda i: (_NB + i, 0)),
            pl.BlockSpec((_BLK, 1), lambda i: (i, 0)),
            pl.BlockSpec((1, 256), lambda i: (0, 0)),
            pl.BlockSpec((256, 128), lambda i: (0, 0)),
            pl.BlockSpec((_BLK, 1), lambda i: (i, 0)),
        ],
        out_specs=pl.BlockSpec((_BLK, 128), lambda i: (i, 0)),
        out_shape=jax.ShapeDtypeStruct((NP, 128), jnp.float32),
    )(agg, agg, nd, b, w, ns)


def _tc_final(a12, nd1, nd2, b, m):
    return pl.pallas_call(
        _final_kernel,
        grid=(_NB,),
        in_specs=[
            pl.BlockSpec((_BLK, 128), lambda i: (i, 0)),
            pl.BlockSpec((_BLK, 128), lambda i: (_NB + i, 0)),
            pl.BlockSpec((_BLK, 1), lambda i: (i, 0)),
            pl.BlockSpec((_BLK, 1), lambda i: (i, 0)),
            pl.BlockSpec((1, 128), lambda i: (0, 0)),
            pl.BlockSpec((_BLK, 1), lambda i: (i, 0)),
        ],
        out_specs=pl.BlockSpec((_BLK, 128), lambda i: (i, 0)),
        out_shape=jax.ShapeDtypeStruct((NP, 128), jnp.float32),
    )(a12, a12, nd1, nd2, b, m)


def kernel(features, edge_index1, edge_index2, mask, W0, b0, W1, b1, W2, b2):
    xp = jnp.pad(features, ((0, NP - N), (0, 0)))
    mp = jnp.pad(mask, (0, NP - N)).reshape(NP, 1)

    edges = jnp.concatenate(
        [edge_index1[0], edge_index1[1], edge_index2[0], edge_index2[1]])
    deg_kernel = _make_deg_kernel()
    degs = deg_kernel(edges)
    norms = _tc_norms(degs).reshape(2, 2, NP, 1)

    aggk = _make_agg_kernel(128)
    z128 = jnp.zeros((NP, 128), jnp.float32)

    hs2s = []
    for g, ei in enumerate((edge_index1, edge_index2)):
        src, dst = ei[0], ei[1]
        src_a = jnp.concatenate([src, src + NP])
        dst_a = jnp.concatenate([dst, dst])
        ns = norms[g, 0]
        nd = norms[g, 1]

        xp_g = xp
        if hs2s:
            xp_g = xp + hs2s[-1][:1, :1] * 0.0

        hs0 = _tc_dense0(xp_g, W0, ns)
        agg0 = aggk(hs0, src_a, dst_a, z128)
        hs1 = _tc_dense(agg0, nd, b0.reshape(1, -1), W1, ns)
        agg1 = aggk(hs1, src_a, dst_a, z128)
        hs2 = _tc_dense_last(agg1, nd, b1.reshape(1, -1), W2, ns)
        hs2s.append(hs2)

    hs2_cat = jnp.concatenate(hs2s, axis=0)
    src_b = jnp.concatenate([edge_index1[0], edge_index2[0] + NP])
    dst_b = jnp.concatenate([edge_index1[1], edge_index2[1]])
    agg2 = aggk(hs2_cat, src_b, dst_b, z128)

    out = _tc_final(agg2, norms[0, 1], norms[1, 1], b2.reshape(1, -1), mp)
    return out[:N]


# speedup vs baseline: 1.0332x; 1.0332x over previous
import jax
import jax.numpy as jnp
from jax import lax
from jax.experimental import pallas as pl
from jax.experimental.pallas import tpu as pltpu
from jax.experimental.pallas import tpu_sc as plsc

N = 10000
NP = 10240
E = 320000

NS = 16
NC = 2
EPT = E // NS
K = 80
NCHUNK = EPT // K
RPT = NP // NS

_mesh = plsc.VectorSubcoreMesh(core_axis_name="c", subcore_axis_name="s")


def _deg_body(edges, degs, isrc_v, idst_v, dsrc_v, ddst_v, pbuf_v, res_v, part_sh):
    cid = lax.axis_index("c")
    sid = lax.axis_index("s")
    zeros16 = jnp.zeros((16,), jnp.float32)
    ones16 = jnp.ones((16,), jnp.float32)

    def zero_body(i, _):
        dsrc_v[pl.ds(i * 16, 16)] = zeros16
        ddst_v[pl.ds(i * 16, 16)] = zeros16
        return 0

    lax.fori_loop(0, NP // 16, zero_body, 0)

    base = cid * (2 * E) + sid * EPT

    def chunk_body(j, _):
        off = base + j * K
        pltpu.sync_copy(edges.at[pl.ds(off, K)], isrc_v)
        pltpu.sync_copy(edges.at[pl.ds(off + E, K)], idst_v)
        for k in range(K // 16):
            i16 = isrc_v[pl.ds(k * 16, 16)]
            plsc.addupdate_scatter(dsrc_v, [i16], ones16)
            j16 = idst_v[pl.ds(k * 16, 16)]
            plsc.addupdate_scatter(ddst_v, [j16], ones16)
        return 0

    lax.fori_loop(0, NCHUNK, chunk_body, 0)

    pltpu.sync_copy(dsrc_v, part_sh.at[0, sid])
    pltpu.sync_copy(ddst_v, part_sh.at[1, sid])
    plsc.subcore_barrier()

    r0 = sid * RPT
    for t in range(2):
        for p in range(NS):
            pltpu.sync_copy(part_sh.at[t, p, pl.ds(r0, RPT)], pbuf_v.at[p])

        def red_loop(q, _):
            s = pbuf_v[0, pl.ds(q * 16, 16)]
            for p in range(1, NS):
                s = s + pbuf_v[p, pl.ds(q * 16, 16)]
            res_v[pl.ds(q * 16, 16)] = s
            return 0

        lax.fori_loop(0, RPT // 16, red_loop, 0)
        pltpu.sync_copy(res_v, degs.at[pl.ds((cid * 2 + t) * NP + r0, RPT)])


def _agg_body(hs, srcoff, dst, zrows, agg,
              isrc_v, idst0_v, idst1_v, rows_v, sem0, sem1, agg_sh):
    cid = lax.axis_index("c")
    sid = lax.axis_index("s")

    r0 = sid * RPT
    pltpu.sync_copy(zrows.at[pl.ds(r0, RPT)], agg_sh.at[pl.ds(r0, RPT)])

    sbase = cid * E + sid * EPT
    pltpu.sync_copy(srcoff.at[pl.ds(sbase, EPT)], isrc_v)
    plsc.subcore_barrier()

    idsts = (idst0_v, idst1_v)
    sems_ = (sem0, sem1)

    def stage_and_fire(j, b):
        pltpu.sync_copy(dst.at[pl.ds(sbase + j * K, K)], idsts[b])
        pltpu.async_copy(hs.at[isrc_v.at[pl.ds(j * K, K)]], rows_v.at[b],
                         sems_[b])

    for b in range(2):
        stage_and_fire(b, b)

    def outer(jj, _):
        for b in range(2):
            j = jj * 2 + b
            pltpu.make_async_copy(hs.at[isrc_v.at[pl.ds(j * K, K)]],
                                  rows_v.at[b], sems_[b]).wait()
            pltpu.sync_copy(rows_v.at[b], agg_sh.at[idsts[b]], add=True)

            @pl.when(j + 2 < NCHUNK)
            def _():
                stage_and_fire(j + 2, b)
        return 0

    lax.fori_loop(0, NCHUNK // 2, outer, 0)
    plsc.subcore_barrier()
    pltpu.sync_copy(agg_sh.at[pl.ds(r0, RPT)], agg.at[pl.ds(cid * NP + r0, RPT)])


def _make_deg_kernel():
    return pl.kernel(
        _deg_body,
        out_type=jax.ShapeDtypeStruct((4 * NP,), jnp.float32),
        mesh=_mesh,
        compiler_params=pltpu.CompilerParams(needs_layout_passes=False),
        scratch_types=[
            pltpu.VMEM((K,), jnp.int32),
            pltpu.VMEM((K,), jnp.int32),
            pltpu.VMEM((NP,), jnp.float32),
            pltpu.VMEM((NP,), jnp.float32),
            pltpu.VMEM((NS, RPT), jnp.float32),
            pltpu.VMEM((RPT,), jnp.float32),
            pltpu.VMEM_SHARED((2, NS, NP), jnp.float32),
        ],
    )


def _make_agg_kernel(fh):
    return pl.kernel(
        _agg_body,
        out_type=jax.ShapeDtypeStruct((NC * NP, fh), jnp.float32),
        mesh=_mesh,
        compiler_params=pltpu.CompilerParams(needs_layout_passes=False),
        scratch_types=[
            pltpu.VMEM((EPT,), jnp.int32),
            pltpu.VMEM((K,), jnp.int32),
            pltpu.VMEM((K,), jnp.int32),
            pltpu.VMEM((2, K, fh), jnp.float32),
            pltpu.SemaphoreType.DMA,
            pltpu.SemaphoreType.DMA,
            pltpu.VMEM_SHARED((NP, fh), jnp.float32),
        ],
    )


def _norms_kernel(deg_ref, out_ref):
    d = deg_ref[...]
    out_ref[...] = lax.rsqrt(jnp.maximum(d, 1.0))


def _dense0_kernel(x_ref, w_ref, ns_ref, out_ref):
    x = x_ref[...]
    w = w_ref[...]
    h = jnp.dot(x, w, preferred_element_type=jnp.float32)
    out_ref[...] = h * ns_ref[...]


def _dense_kernel(a0_ref, a1_ref, nd_ref, b_ref, w_ref, ns_ref, out_ref):
    x = jnp.concatenate([a0_ref[...], a1_ref[...]], axis=-1)
    x = jnp.maximum(x * nd_ref[...] + b_ref[...], 0.0)
    h = jnp.dot(x, w_ref[...], preferred_element_type=jnp.float32)
    out_ref[...] = h * ns_ref[...]


def _final_kernel(a1_ref, a2_ref, nd1_ref, nd2_ref, b_ref, m_ref, out_ref):
    o1 = a1_ref[...] * nd1_ref[...] + b_ref[...]
    o2 = a2_ref[...] * nd2_ref[...] + b_ref[...]
    out_ref[...] = jnp.where(m_ref[...] == 0, o1, o2)


_BLK = 640
_NB = NP // _BLK


def _tc_norms(degs):
    return pl.pallas_call(
        _norms_kernel,
        out_shape=jax.ShapeDtypeStruct((4, NP), jnp.float32),
    )(degs.reshape(4, NP))


def _tc_dense0(x, w, ns):
    fh = w.shape[1] // 2
    return pl.pallas_call(
        _dense0_kernel,
        grid=(_NB, 2),
        in_specs=[
            pl.BlockSpec((_BLK, 128), lambda i, c: (i, 0)),
            pl.BlockSpec((128, fh), lambda i, c: (0, c)),
            pl.BlockSpec((_BLK, 1), lambda i, c: (i, 0)),
        ],
        out_specs=pl.BlockSpec((_BLK, fh), lambda i, c: (c * _NB + i, 0)),
        out_shape=jax.ShapeDtypeStruct((2 * NP, fh), jnp.float32),
    )(x, w, ns)


def _tc_dense(agg, nd, b, w, ns):
    fh = w.shape[1] // 2
    return pl.pallas_call(
        _dense_kernel,
        grid=(_NB, 2),
        in_specs=[
            pl.BlockSpec((_BLK, 128), lambda i, c: (i, 0)),
            pl.BlockSpec((_BLK, 128), lambda i, c: (_NB + i, 0)),
            pl.BlockSpec((_BLK, 1), lambda i, c: (i, 0)),
            pl.BlockSpec((1, 256), lambda i, c: (0, 0)),
            pl.BlockSpec((256, fh), lambda i, c: (0, c)),
            pl.BlockSpec((_BLK, 1), lambda i, c: (i, 0)),
        ],
        out_specs=pl.BlockSpec((_BLK, fh), lambda i, c: (c * _NB + i, 0)),
        out_shape=jax.ShapeDtypeStruct((2 * NP, fh), jnp.float32),
    )(agg, agg, nd, b, w, ns)


def _tc_dense_last(agg, nd, b, w, ns):
    return pl.pallas_call(
        _dense_kernel,
        grid=(_NB,),
        in_specs=[
            pl.BlockSpec((_BLK, 128), lambda i: (i, 0)),
            pl.BlockSpec((_BLK, 128), lambda i: (_NB + i, 0)),
            pl.BlockSpec((_BLK, 1), lambda i: (i, 0)),
            pl.BlockSpec((1, 256), lambda i: (0, 0)),
            pl.BlockSpec((256, 128), lambda i: (0, 0)),
            pl.BlockSpec((_BLK, 1), lambda i: (i, 0)),
        ],
        out_specs=pl.BlockSpec((_BLK, 128), lambda i: (i, 0)),
        out_shape=jax.ShapeDtypeStruct((NP, 128), jnp.float32),
    )(agg, agg, nd, b, w, ns)


def _tc_final(a12, nd1, nd2, b, m):
    return pl.pallas_call(
        _final_kernel,
        grid=(_NB,),
        in_specs=[
            pl.BlockSpec((_BLK, 128), lambda i: (i, 0)),
            pl.BlockSpec((_BLK, 128), lambda i: (_NB + i, 0)),
            pl.BlockSpec((_BLK, 1), lambda i: (i, 0)),
            pl.BlockSpec((_BLK, 1), lambda i: (i, 0)),
            pl.BlockSpec((1, 128), lambda i: (0, 0)),
            pl.BlockSpec((_BLK, 1), lambda i: (i, 0)),
        ],
        out_specs=pl.BlockSpec((_BLK, 128), lambda i: (i, 0)),
        out_shape=jax.ShapeDtypeStruct((NP, 128), jnp.float32),
    )(a12, a12, nd1, nd2, b, m)


def kernel(features, edge_index1, edge_index2, mask, W0, b0, W1, b1, W2, b2):
    xp = jnp.pad(features, ((0, NP - N), (0, 0)))
    mp = jnp.pad(mask, (0, NP - N)).reshape(NP, 1)

    edges = jnp.concatenate(
        [edge_index1[0], edge_index1[1], edge_index2[0], edge_index2[1]])
    deg_kernel = _make_deg_kernel()
    degs = deg_kernel(edges)
    norms = _tc_norms(degs).reshape(2, 2, NP, 1)

    aggk = _make_agg_kernel(128)
    z128 = jnp.zeros((NP, 128), jnp.float32)

    hs2s = []
    for g, ei in enumerate((edge_index1, edge_index2)):
        src, dst = ei[0], ei[1]
        src_a = jnp.concatenate([src, src + NP])
        dst_a = jnp.concatenate([dst, dst])
        ns = norms[g, 0]
        nd = norms[g, 1]

        xp_g = xp
        if hs2s:
            xp_g = xp + hs2s[-1][:1, :1] * 0.0

        hs0 = _tc_dense0(xp_g, W0, ns)
        agg0 = aggk(hs0, src_a, dst_a, z128)
        hs1 = _tc_dense(agg0, nd, b0.reshape(1, -1), W1, ns)
        agg1 = aggk(hs1, src_a, dst_a, z128)
        hs2 = _tc_dense_last(agg1, nd, b1.reshape(1, -1), W2, ns)
        hs2s.append(hs2)

    hs2_cat = jnp.concatenate(hs2s, axis=0)
    src_b = jnp.concatenate([edge_index1[0], edge_index2[0] + NP])
    dst_b = jnp.concatenate([edge_index1[1], edge_index2[1]])
    agg2 = aggk(hs2_cat, src_b, dst_b, z128)

    out = _tc_final(agg2, norms[0, 1], norms[1, 1], b2.reshape(1, -1), mp)
    return out[:N]


# speedup vs baseline: 1.1638x; 1.1264x over previous
import jax
import jax.numpy as jnp
from jax import lax
from jax.experimental import pallas as pl
from jax.experimental.pallas import tpu as pltpu
from jax.experimental.pallas import tpu_sc as plsc

N = 10000
NP = 10240
E = 320000

NS = 16
NC = 2
EPT = E // NS
K = 80
NCHUNK = EPT // K
RPT = NP // NS

_mesh = plsc.VectorSubcoreMesh(core_axis_name="c", subcore_axis_name="s")


def _deg_body(edges, degs, isrc_v, idst_v, dsrc_v, ddst_v, pbuf_v, res_v, part_sh):
    cid = lax.axis_index("c")
    sid = lax.axis_index("s")
    zeros16 = jnp.zeros((16,), jnp.float32)
    ones16 = jnp.ones((16,), jnp.float32)

    def zero_body(i, _):
        dsrc_v[pl.ds(i * 16, 16)] = zeros16
        ddst_v[pl.ds(i * 16, 16)] = zeros16
        return 0

    lax.fori_loop(0, NP // 16, zero_body, 0)

    base = cid * (2 * E) + sid * EPT
    pltpu.sync_copy(edges.at[pl.ds(base, EPT)], isrc_v)
    pltpu.sync_copy(edges.at[pl.ds(base + E, EPT)], idst_v)

    def chunk_body(q, _):
        i16 = isrc_v[pl.ds(q * 16, 16)]
        plsc.addupdate_scatter(dsrc_v, [i16], ones16)
        j16 = idst_v[pl.ds(q * 16, 16)]
        plsc.addupdate_scatter(ddst_v, [j16], ones16)
        return 0

    lax.fori_loop(0, EPT // 16, chunk_body, 0)

    pltpu.sync_copy(dsrc_v, part_sh.at[0, sid])
    pltpu.sync_copy(ddst_v, part_sh.at[1, sid])
    plsc.subcore_barrier()

    r0 = sid * RPT
    for t in range(2):
        for p in range(NS):
            pltpu.sync_copy(part_sh.at[t, p, pl.ds(r0, RPT)], pbuf_v.at[p])

        def red_loop(q, _):
            s = pbuf_v[0, pl.ds(q * 16, 16)]
            for p in range(1, NS):
                s = s + pbuf_v[p, pl.ds(q * 16, 16)]
            res_v[pl.ds(q * 16, 16)] = s
            return 0

        lax.fori_loop(0, RPT // 16, red_loop, 0)
        pltpu.sync_copy(res_v, degs.at[pl.ds((cid * 2 + t) * NP + r0, RPT)])


def _agg_body(hs, srcoff, dst, zrows, agg,
              isrc_v, idst0_v, idst1_v, rows_v, sem0, sem1, agg_sh):
    cid = lax.axis_index("c")
    sid = lax.axis_index("s")

    r0 = sid * RPT
    pltpu.sync_copy(zrows.at[pl.ds(r0, RPT)], agg_sh.at[pl.ds(r0, RPT)])

    sbase = cid * E + sid * EPT
    pltpu.sync_copy(srcoff.at[pl.ds(sbase, EPT)], isrc_v)
    plsc.subcore_barrier()

    idsts = (idst0_v, idst1_v)
    sems_ = (sem0, sem1)

    def stage_and_fire(j, b):
        pltpu.sync_copy(dst.at[pl.ds(sbase + j * K, K)], idsts[b])
        pltpu.async_copy(hs.at[isrc_v.at[pl.ds(j * K, K)]], rows_v.at[b],
                         sems_[b])

    for b in range(2):
        stage_and_fire(b, b)

    def outer(jj, _):
        for b in range(2):
            j = jj * 2 + b
            pltpu.make_async_copy(hs.at[isrc_v.at[pl.ds(j * K, K)]],
                                  rows_v.at[b], sems_[b]).wait()
            pltpu.sync_copy(rows_v.at[b], agg_sh.at[idsts[b]], add=True)

            @pl.when(j + 2 < NCHUNK)
            def _():
                stage_and_fire(j + 2, b)
        return 0

    lax.fori_loop(0, NCHUNK // 2, outer, 0)
    plsc.subcore_barrier()
    pltpu.sync_copy(agg_sh.at[pl.ds(r0, RPT)], agg.at[pl.ds(cid * NP + r0, RPT)])


def _make_deg_kernel():
    return pl.kernel(
        _deg_body,
        out_type=jax.ShapeDtypeStruct((4 * NP,), jnp.float32),
        mesh=_mesh,
        compiler_params=pltpu.CompilerParams(needs_layout_passes=False),
        scratch_types=[
            pltpu.VMEM((EPT,), jnp.int32),
            pltpu.VMEM((EPT,), jnp.int32),
            pltpu.VMEM((NP,), jnp.float32),
            pltpu.VMEM((NP,), jnp.float32),
            pltpu.VMEM((NS, RPT), jnp.float32),
            pltpu.VMEM((RPT,), jnp.float32),
            pltpu.VMEM_SHARED((2, NS, NP), jnp.float32),
        ],
    )


def _make_agg_kernel(fh):
    return pl.kernel(
        _agg_body,
        out_type=jax.ShapeDtypeStruct((NC * NP, fh), jnp.float32),
        mesh=_mesh,
        compiler_params=pltpu.CompilerParams(needs_layout_passes=False),
        scratch_types=[
            pltpu.VMEM((EPT,), jnp.int32),
            pltpu.VMEM((K,), jnp.int32),
            pltpu.VMEM((K,), jnp.int32),
            pltpu.VMEM((2, K, fh), jnp.float32),
            pltpu.SemaphoreType.DMA,
            pltpu.SemaphoreType.DMA,
            pltpu.VMEM_SHARED((NP, fh), jnp.float32),
        ],
    )


def _norms_kernel(deg_ref, out_ref):
    d = deg_ref[...]
    out_ref[...] = lax.rsqrt(jnp.maximum(d, 1.0))


def _dense0_kernel(x_ref, w_ref, ns_ref, out_ref):
    x = x_ref[...]
    w = w_ref[...]
    h = jnp.dot(x, w, preferred_element_type=jnp.float32)
    out_ref[...] = h * ns_ref[...]


def _dense_kernel(a0_ref, a1_ref, nd_ref, b_ref, w_ref, ns_ref, out_ref):
    x = jnp.concatenate([a0_ref[...], a1_ref[...]], axis=-1)
    x = jnp.maximum(x * nd_ref[...] + b_ref[...], 0.0)
    h = jnp.dot(x, w_ref[...], preferred_element_type=jnp.float32)
    out_ref[...] = h * ns_ref[...]


def _final_kernel(a1_ref, a2_ref, nd1_ref, nd2_ref, b_ref, m_ref, out_ref):
    o1 = a1_ref[...] * nd1_ref[...] + b_ref[...]
    o2 = a2_ref[...] * nd2_ref[...] + b_ref[...]
    out_ref[...] = jnp.where(m_ref[...] == 0, o1, o2)


_BLK = 640
_NB = NP // _BLK


def _tc_norms(degs):
    return pl.pallas_call(
        _norms_kernel,
        out_shape=jax.ShapeDtypeStruct((4, NP), jnp.float32),
    )(degs.reshape(4, NP))


def _tc_dense0(x, w, ns):
    fh = w.shape[1] // 2
    return pl.pallas_call(
        _dense0_kernel,
        grid=(_NB, 2),
        in_specs=[
            pl.BlockSpec((_BLK, 128), lambda i, c: (i, 0)),
            pl.BlockSpec((128, fh), lambda i, c: (0, c)),
            pl.BlockSpec((_BLK, 1), lambda i, c: (i, 0)),
        ],
        out_specs=pl.BlockSpec((_BLK, fh), lambda i, c: (c * _NB + i, 0)),
        out_shape=jax.ShapeDtypeStruct((2 * NP, fh), jnp.float32),
    )(x, w, ns)


def _tc_dense(agg, nd, b, w, ns):
    fh = w.shape[1] // 2
    return pl.pallas_call(
        _dense_kernel,
        grid=(_NB, 2),
        in_specs=[
            pl.BlockSpec((_BLK, 128), lambda i, c: (i, 0)),
            pl.BlockSpec((_BLK, 128), lambda i, c: (_NB + i, 0)),
            pl.BlockSpec((_BLK, 1), lambda i, c: (i, 0)),
            pl.BlockSpec((1, 256), lambda i, c: (0, 0)),
            pl.BlockSpec((256, fh), lambda i, c: (0, c)),
            pl.BlockSpec((_BLK, 1), lambda i, c: (i, 0)),
        ],
        out_specs=pl.BlockSpec((_BLK, fh), lambda i, c: (c * _NB + i, 0)),
        out_shape=jax.ShapeDtypeStruct((2 * NP, fh), jnp.float32),
    )(agg, agg, nd, b, w, ns)


def _tc_dense_last(agg, nd, b, w, ns):
    return pl.pallas_call(
        _dense_kernel,
        grid=(_NB,),
        in_specs=[
            pl.BlockSpec((_BLK, 128), lambda i: (i, 0)),
            pl.BlockSpec((_BLK, 128), lambda i: (_NB + i, 0)),
            pl.BlockSpec((_BLK, 1), lambda i: (i, 0)),
            pl.BlockSpec((1, 256), lambda i: (0, 0)),
            pl.BlockSpec((256, 128), lambda i: (0, 0)),
            pl.BlockSpec((_BLK, 1), lambda i: (i, 0)),
        ],
        out_specs=pl.BlockSpec((_BLK, 128), lambda i: (i, 0)),
        out_shape=jax.ShapeDtypeStruct((NP, 128), jnp.float32),
    )(agg, agg, nd, b, w, ns)


def _tc_final(a12, nd1, nd2, b, m):
    return pl.pallas_call(
        _final_kernel,
        grid=(_NB,),
        in_specs=[
            pl.BlockSpec((_BLK, 128), lambda i: (i, 0)),
            pl.BlockSpec((_BLK, 128), lambda i: (_NB + i, 0)),
            pl.BlockSpec((_BLK, 1), lambda i: (i, 0)),
            pl.BlockSpec((_BLK, 1), lambda i: (i, 0)),
            pl.BlockSpec((1, 128), lambda i: (0, 0)),
            pl.BlockSpec((_BLK, 1), lambda i: (i, 0)),
        ],
        out_specs=pl.BlockSpec((_BLK, 128), lambda i: (i, 0)),
        out_shape=jax.ShapeDtypeStruct((NP, 128), jnp.float32),
    )(a12, a12, nd1, nd2, b, m)


def kernel(features, edge_index1, edge_index2, mask, W0, b0, W1, b1, W2, b2):
    xp = jnp.pad(features, ((0, NP - N), (0, 0)))
    mp = jnp.pad(mask, (0, NP - N)).reshape(NP, 1)

    edges = jnp.concatenate(
        [edge_index1[0], edge_index1[1], edge_index2[0], edge_index2[1]])
    deg_kernel = _make_deg_kernel()
    degs = deg_kernel(edges)
    norms = _tc_norms(degs).reshape(2, 2, NP, 1)

    aggk = _make_agg_kernel(128)
    z128 = jnp.zeros((NP, 128), jnp.float32)

    hs2s = []
    for g, ei in enumerate((edge_index1, edge_index2)):
        src, dst = ei[0], ei[1]
        src_a = jnp.concatenate([src, src + NP])
        dst_a = jnp.concatenate([dst, dst])
        ns = norms[g, 0]
        nd = norms[g, 1]

        xp_g = xp
        if hs2s:
            xp_g = xp + hs2s[-1][:1, :1] * 0.0

        hs0 = _tc_dense0(xp_g, W0, ns)
        agg0 = aggk(hs0, src_a, dst_a, z128)
        hs1 = _tc_dense(agg0, nd, b0.reshape(1, -1), W1, ns)
        agg1 = aggk(hs1, src_a, dst_a, z128)
        hs2 = _tc_dense_last(agg1, nd, b1.reshape(1, -1), W2, ns)
        hs2s.append(hs2)

    hs2_cat = jnp.concatenate(hs2s, axis=0)
    src_b = jnp.concatenate([edge_index1[0], edge_index2[0] + NP])
    dst_b = jnp.concatenate([edge_index1[1], edge_index2[1]])
    agg2 = aggk(hs2_cat, src_b, dst_b, z128)

    out = _tc_final(agg2, norms[0, 1], norms[1, 1], b2.reshape(1, -1), mp)
    return out[:N]


# speedup vs baseline: 1.2373x; 1.0632x over previous
import jax
import jax.numpy as jnp
from jax import lax
from jax.experimental import pallas as pl
from jax.experimental.pallas import tpu as pltpu
from jax.experimental.pallas import tpu_sc as plsc

N = 10000
NP = 10240
E = 320000

NS = 16
NC = 2
EPT = E // NS
K = 80
NCHUNK = EPT // K
RPT = NP // NS

_mesh = plsc.VectorSubcoreMesh(core_axis_name="c", subcore_axis_name="s")


def _deg_body(edges, degs, isrc_v, idst_v, dsrc_v, ddst_v, pbuf_v, res_v, part_sh):
    cid = lax.axis_index("c")
    sid = lax.axis_index("s")
    zeros16 = jnp.zeros((16,), jnp.float32)
    ones16 = jnp.ones((16,), jnp.float32)

    def zero_body(i, _):
        dsrc_v[pl.ds(i * 16, 16)] = zeros16
        ddst_v[pl.ds(i * 16, 16)] = zeros16
        return 0

    lax.fori_loop(0, NP // 16, zero_body, 0)

    base = cid * (2 * E) + sid * EPT
    pltpu.sync_copy(edges.at[pl.ds(base, EPT)], isrc_v)
    pltpu.sync_copy(edges.at[pl.ds(base + E, EPT)], idst_v)

    def chunk_body(q, _):
        i16 = isrc_v[pl.ds(q * 16, 16)]
        plsc.addupdate_scatter(dsrc_v, [i16], ones16)
        j16 = idst_v[pl.ds(q * 16, 16)]
        plsc.addupdate_scatter(ddst_v, [j16], ones16)
        return 0

    lax.fori_loop(0, EPT // 16, chunk_body, 0)

    pltpu.sync_copy(dsrc_v, part_sh.at[0, sid])
    pltpu.sync_copy(ddst_v, part_sh.at[1, sid])
    plsc.subcore_barrier()

    r0 = sid * RPT
    for t in range(2):
        for p in range(NS):
            pltpu.sync_copy(part_sh.at[t, p, pl.ds(r0, RPT)], pbuf_v.at[p])

        def red_loop(q, _):
            s = pbuf_v[0, pl.ds(q * 16, 16)]
            for p in range(1, NS):
                s = s + pbuf_v[p, pl.ds(q * 16, 16)]
            res_v[pl.ds(q * 16, 16)] = s
            return 0

        lax.fori_loop(0, RPT // 16, red_loop, 0)
        pltpu.sync_copy(res_v, degs.at[pl.ds((cid * 2 + t) * NP + r0, RPT)])


def _agg_body(hs, srcoff, dst, zrows, agg,
              isrc_v, idst0_v, idst1_v, rows_v, sem0, sem1, agg_sh):
    cid = lax.axis_index("c")
    sid = lax.axis_index("s")

    r0 = sid * RPT
    pltpu.sync_copy(zrows.at[pl.ds(r0, RPT)], agg_sh.at[pl.ds(r0, RPT)])

    sbase = cid * E + sid * EPT
    pltpu.sync_copy(srcoff.at[pl.ds(sbase, EPT)], isrc_v)
    plsc.subcore_barrier()

    idsts = (idst0_v, idst1_v)
    sems_ = (sem0, sem1)

    def stage_and_fire(j, b):
        pltpu.sync_copy(dst.at[pl.ds(sbase + j * K, K)], idsts[b])
        pltpu.async_copy(hs.at[isrc_v.at[pl.ds(j * K, K)]], rows_v.at[b],
                         sems_[b])

    for b in range(2):
        stage_and_fire(b, b)

    def outer(jj, _):
        for b in range(2):
            j = jj * 2 + b
            pltpu.make_async_copy(hs.at[isrc_v.at[pl.ds(j * K, K)]],
                                  rows_v.at[b], sems_[b]).wait()
            pltpu.sync_copy(rows_v.at[b], agg_sh.at[idsts[b]], add=True)

            @pl.when(j + 2 < NCHUNK)
            def _():
                stage_and_fire(j + 2, b)
        return 0

    lax.fori_loop(0, NCHUNK // 2, outer, 0)
    plsc.subcore_barrier()
    pltpu.sync_copy(agg_sh.at[pl.ds(r0, RPT)], agg.at[pl.ds(cid * NP + r0, RPT)])


def _make_deg_kernel():
    return pl.kernel(
        _deg_body,
        out_type=jax.ShapeDtypeStruct((4 * NP,), jnp.float32),
        mesh=_mesh,
        compiler_params=pltpu.CompilerParams(needs_layout_passes=False),
        scratch_types=[
            pltpu.VMEM((EPT,), jnp.int32),
            pltpu.VMEM((EPT,), jnp.int32),
            pltpu.VMEM((NP,), jnp.float32),
            pltpu.VMEM((NP,), jnp.float32),
            pltpu.VMEM((NS, RPT), jnp.float32),
            pltpu.VMEM((RPT,), jnp.float32),
            pltpu.VMEM_SHARED((2, NS, NP), jnp.float32),
        ],
    )


def _make_agg_kernel(fh):
    return pl.kernel(
        _agg_body,
        out_type=jax.ShapeDtypeStruct((NC * NP, fh), jnp.float32),
        mesh=_mesh,
        compiler_params=pltpu.CompilerParams(needs_layout_passes=False),
        scratch_types=[
            pltpu.VMEM((EPT,), jnp.int32),
            pltpu.VMEM((K,), jnp.int32),
            pltpu.VMEM((K,), jnp.int32),
            pltpu.VMEM((2, K, fh), jnp.float32),
            pltpu.SemaphoreType.DMA,
            pltpu.SemaphoreType.DMA,
            pltpu.VMEM_SHARED((NP, fh), jnp.float32),
        ],
    )


def _norms_kernel(deg_ref, out_ref):
    d = deg_ref[...]
    out_ref[...] = lax.rsqrt(jnp.maximum(d, 1.0))


def _dense0_kernel(x_ref, w_ref, ns_ref, out_ref):
    x = x_ref[...]
    w = w_ref[...]
    h = jnp.dot(x, w, preferred_element_type=jnp.float32)
    out_ref[...] = h * ns_ref[...]


def _dense_kernel(a0_ref, a1_ref, nd_ref, b_ref, w_ref, ns_ref, out_ref):
    x = jnp.concatenate([a0_ref[...], a1_ref[...]], axis=-1)
    x = jnp.maximum(x * nd_ref[...] + b_ref[...], 0.0)
    h = jnp.dot(x, w_ref[...], preferred_element_type=jnp.float32)
    out_ref[...] = h * ns_ref[...]


def _final_kernel(a1_ref, a2_ref, nd1_ref, nd2_ref, b_ref, m_ref, out_ref):
    o1 = a1_ref[...] * nd1_ref[...] + b_ref[...]
    o2 = a2_ref[...] * nd2_ref[...] + b_ref[...]
    out_ref[...] = jnp.where(m_ref[...] == 0, o1, o2)


_BLK = 640
_NB = NP // _BLK


def _tc_norms(degs):
    return pl.pallas_call(
        _norms_kernel,
        out_shape=jax.ShapeDtypeStruct((4, NP), jnp.float32),
    )(degs.reshape(4, NP))


def _tc_dense0(x, w, ns):
    fh = w.shape[1] // 2
    return pl.pallas_call(
        _dense0_kernel,
        grid=(_NB, 2),
        in_specs=[
            pl.BlockSpec((_BLK, 128), lambda i, c: (i, 0)),
            pl.BlockSpec((128, fh), lambda i, c: (0, c)),
            pl.BlockSpec((_BLK, 1), lambda i, c: (i, 0)),
        ],
        out_specs=pl.BlockSpec((_BLK, fh), lambda i, c: (c * _NB + i, 0)),
        out_shape=jax.ShapeDtypeStruct((2 * NP, fh), jnp.float32),
    )(x, w, ns)


def _tc_dense(agg, nd, b, w, ns):
    fh = w.shape[1] // 2
    return pl.pallas_call(
        _dense_kernel,
        grid=(_NB, 2),
        in_specs=[
            pl.BlockSpec((_BLK, 128), lambda i, c: (i, 0)),
            pl.BlockSpec((_BLK, 128), lambda i, c: (_NB + i, 0)),
            pl.BlockSpec((_BLK, 1), lambda i, c: (i, 0)),
            pl.BlockSpec((1, 256), lambda i, c: (0, 0)),
            pl.BlockSpec((256, fh), lambda i, c: (0, c)),
            pl.BlockSpec((_BLK, 1), lambda i, c: (i, 0)),
        ],
        out_specs=pl.BlockSpec((_BLK, fh), lambda i, c: (c * _NB + i, 0)),
        out_shape=jax.ShapeDtypeStruct((2 * NP, fh), jnp.float32),
    )(agg, agg, nd, b, w, ns)


def _tc_dense_last(agg, nd, b, w, ns):
    return pl.pallas_call(
        _dense_kernel,
        grid=(_NB,),
        in_specs=[
            pl.BlockSpec((_BLK, 128), lambda i: (i, 0)),
            pl.BlockSpec((_BLK, 128), lambda i: (_NB + i, 0)),
            pl.BlockSpec((_BLK, 1), lambda i: (i, 0)),
            pl.BlockSpec((1, 256), lambda i: (0, 0)),
            pl.BlockSpec((256, 128), lambda i: (0, 0)),
            pl.BlockSpec((_BLK, 1), lambda i: (i, 0)),
        ],
        out_specs=pl.BlockSpec((_BLK, 128), lambda i: (i, 0)),
        out_shape=jax.ShapeDtypeStruct((NP, 128), jnp.float32),
    )(agg, agg, nd, b, w, ns)


def _tc_final(a12, nd1, nd2, b, m):
    return pl.pallas_call(
        _final_kernel,
        grid=(_NB,),
        in_specs=[
            pl.BlockSpec((_BLK, 128), lambda i: (i, 0)),
            pl.BlockSpec((_BLK, 128), lambda i: (_NB + i, 0)),
            pl.BlockSpec((_BLK, 1), lambda i: (i, 0)),
            pl.BlockSpec((_BLK, 1), lambda i: (i, 0)),
            pl.BlockSpec((1, 128), lambda i: (0, 0)),
            pl.BlockSpec((_BLK, 1), lambda i: (i, 0)),
        ],
        out_specs=pl.BlockSpec((_BLK, 128), lambda i: (i, 0)),
        out_shape=jax.ShapeDtypeStruct((NP, 128), jnp.float32),
    )(a12, a12, nd1, nd2, b, m)


def kernel(features, edge_index1, edge_index2, mask, W0, b0, W1, b1, W2, b2):
    xp = jnp.pad(features, ((0, NP - N), (0, 0)))
    mp = jnp.pad(mask, (0, NP - N)).reshape(NP, 1)

    edges = jnp.concatenate(
        [edge_index1[0], edge_index1[1], edge_index2[0], edge_index2[1]])
    deg_kernel = _make_deg_kernel()
    degs = deg_kernel(edges)
    norms = _tc_norms(degs).reshape(2, 2, NP, 1)

    aggk = _make_agg_kernel(128)
    z128 = jnp.zeros((NP, 128), jnp.float32)

    hs2s = []
    for g, ei in enumerate((edge_index1, edge_index2)):
        src, dst = ei[0], ei[1]
        src_a = jnp.concatenate([src, src + NP])
        dst_a = jnp.concatenate([dst, dst])
        ns = norms[g, 0]
        nd = norms[g, 1]

        hs0 = _tc_dense0(xp, W0, ns)
        agg0 = aggk(hs0, src_a, dst_a, z128)
        hs1 = _tc_dense(agg0, nd, b0.reshape(1, -1), W1, ns)
        agg1 = aggk(hs1, src_a, dst_a, z128)
        hs2 = _tc_dense_last(agg1, nd, b1.reshape(1, -1), W2, ns)
        hs2s.append(hs2)

    hs2_cat = jnp.concatenate(hs2s, axis=0)
    src_b = jnp.concatenate([edge_index1[0], edge_index2[0] + NP])
    dst_b = jnp.concatenate([edge_index1[1], edge_index2[1]])
    agg2 = aggk(hs2_cat, src_b, dst_b, z128)

    out = _tc_final(agg2, norms[0, 1], norms[1, 1], b2.reshape(1, -1), mp)
    return out[:N]
